# baseline (speedup 1.0000x reference)
import functools

import jax
import jax.numpy as jnp
from jax import lax
from jax.experimental import pallas as pl
from jax.experimental.pallas import tpu as pltpu
from jax.experimental.pallas import tpu_sc as plsc

D = 64
NT = 15
B = 1024
P = 10000
P_PAD = 10240
NC, NS = 2, 16
NW = NC * NS
CB = B // NW
PB = P_PAD // NW
CHUNK = 64
PN = 1024


def _sc_gather_body(cidx, pidx, *refs):
    (ctab, ptab, t1, t2, t3, t4, t5, t6, t7, t8, t9, t10, t11, t12, t13, t14,
     cfeat, pfeat, cidx_v, pidx_v, rows_v, sem) = refs
    shared = [t1, t2, t3, t4, t5, t6, t7, t8, t9, t10, t11, t12, t13, t14]
    ctabs = [ctab] + shared
    ptabs = [ptab] + shared

    wid = lax.axis_index("s") * NC + lax.axis_index("c")

    pltpu.sync_copy(cidx.at[wid], cidx_v)
    pltpu.sync_copy(pidx.at[wid], pidx_v)

    gops = [
        pltpu.async_copy(ctabs[t].at[cidx_v.at[t]], rows_v.at[t, pl.ds(0, CB)], sem)
        for t in range(NT)
    ]
    for op in gops:
        op.wait()
    sops = [
        pltpu.async_copy(rows_v.at[t, pl.ds(0, CB)],
                         cfeat.at[t, pl.ds(wid * CB, CB)], sem)
        for t in range(NT)
    ]
    for op in sops:
        op.wait()

    def chunk_body(ci, carry):
        off = ci * CHUNK
        g = [
            pltpu.async_copy(ptabs[t].at[pidx_v.at[t, pl.ds(off, CHUNK)]],
                             rows_v.at[t], sem)
            for t in range(NT)
        ]
        for op in g:
            op.wait()
        s = [
            pltpu.async_copy(rows_v.at[t],
                             pfeat.at[t, pl.ds(wid * PB + off, CHUNK)], sem)
            for t in range(NT)
        ]
        for op in s:
            op.wait()
        return carry

    lax.fori_loop(0, PB // CHUNK, chunk_body, 0)


_sc_gather = functools.partial(
    pl.kernel,
    out_type=(jax.ShapeDtypeStruct((NT, B, D), jnp.float32),
              jax.ShapeDtypeStruct((NT, P_PAD, D), jnp.float32)),
    mesh=plsc.VectorSubcoreMesh(core_axis_name="c", subcore_axis_name="s",
                                num_cores=NC, num_subcores=NS),
    scratch_types=[
        pltpu.VMEM((NT, CB), jnp.int32),
        pltpu.VMEM((NT, PB), jnp.int32),
        pltpu.VMEM((NT, CHUNK, D), jnp.float32),
        pltpu.SemaphoreType.DMA,
    ],
    compiler_params=pltpu.CompilerParams(use_tc_tiling_on_sc=False),
)(_sc_gather_body)


def _mm_body(c_ref, p_ref, o_ref):
    acc = lax.dot_general(
        c_ref[0], p_ref[0],
        dimension_numbers=(((1,), (1,)), ((), ())),
        preferred_element_type=jnp.float32,
        precision=lax.Precision.HIGHEST,
    )
    for t in range(1, NT):
        acc += lax.dot_general(
            c_ref[t], p_ref[t],
            dimension_numbers=(((1,), (1,)), ((), ())),
            preferred_element_type=jnp.float32,
            precision=lax.Precision.HIGHEST,
        )
    o_ref[...] = acc


def _matmul(cfeat, pfeat):
    return pl.pallas_call(
        _mm_body,
        grid=(P_PAD // PN,),
        in_specs=[
            pl.BlockSpec((NT, B, D), lambda j: (0, 0, 0)),
            pl.BlockSpec((NT, PN, D), lambda j: (0, j, 0)),
        ],
        out_specs=pl.BlockSpec((B, PN), lambda j: (0, j)),
        out_shape=jax.ShapeDtypeStruct((B, P), jnp.float32),
    )(cfeat, pfeat)


def kernel(Customer_data, Product_data, customer_table, product_table,
           price_table, age_table, colour_table, department_table,
           prod_name_table, sales_channel_table, season_table, day_table,
           month_table, year_table, club_table, fashion_table, postal_table,
           graphical_table):
    cidx = Customer_data.astype(jnp.int32).T.reshape(NT, NW, CB)
    cidx = cidx.transpose(1, 0, 2)
    pidx = jnp.pad(Product_data.astype(jnp.int32),
                   ((0, P_PAD - P), (0, 0))).T.reshape(NT, NW, PB)
    pidx = pidx.transpose(1, 0, 2)
    tables = (customer_table, product_table, club_table, fashion_table,
              age_table, postal_table, price_table, sales_channel_table,
              season_table, day_table, month_table, year_table,
              prod_name_table, graphical_table, colour_table, department_table)
    cfeat, pfeat = _sc_gather(cidx, pidx, *tables)
    return _matmul(cfeat, pfeat)


# speedup vs baseline: 5.2703x; 5.2703x over previous
import functools

import jax
import jax.numpy as jnp
from jax import lax
from jax.experimental import pallas as pl
from jax.experimental.pallas import tpu as pltpu
from jax.experimental.pallas import tpu_sc as plsc

D = 64
NT = 15
B = 1024
P = 10000
P_PAD = 10240
NC, NS = 2, 16
NW = NC * NS
CB = B // NW
PB = P_PAD // NW
CHUNK = 80
NCH = PB // CHUNK
PN = 1024

NBIG_P = 3
NBIG_C = 2

SMALL_COLS = (1, 2, 3, 5, 6, 7, 8, 9, 10, 12, 13, 14)
SMALL_V = (5, 5, 113, 1002, 3, 5, 32, 13, 11, 33, 65, 257)
NSM = len(SMALL_COLS)


def _sc_gather_body(cidx, pidx, ptab, postal, pname,
                    cfeat, pfeat, cidx_v, pidx_v, crows_v, prows_v, gsem, ssem):
    ptabs = [ptab, postal, pname]
    ctabs = [postal, pname]

    wid = lax.axis_index("s") * NC + lax.axis_index("c")

    pltpu.sync_copy(cidx.at[wid], cidx_v)
    pltpu.sync_copy(pidx.at[wid], pidx_v)

    cg = [
        pltpu.async_copy(ctabs[t].at[cidx_v.at[t]], crows_v.at[t], gsem)
        for t in range(NBIG_C)
    ]
    pg = []
    for ci in range(NCH):
        off = ci * CHUNK
        pg.append([
            pltpu.async_copy(ptabs[t].at[pidx_v.at[t, pl.ds(off, CHUNK)]],
                             prows_v.at[ci, t], gsem)
            for t in range(NBIG_P)
        ])
    sops = []
    for op in cg:
        op.wait()
    for t in range(NBIG_C):
        sops.append(pltpu.async_copy(
            crows_v.at[t], cfeat.at[t, pl.ds(wid * CB, CB)], ssem))
    for ci in range(NCH):
        off = ci * CHUNK
        for op in pg[ci]:
            op.wait()
        for t in range(NBIG_P):
            sops.append(pltpu.async_copy(
                prows_v.at[ci, t],
                pfeat.at[t, pl.ds(wid * PB + off, CHUNK)], ssem))
    for op in sops:
        op.wait()


_sc_gather = functools.partial(
    pl.kernel,
    out_type=(jax.ShapeDtypeStruct((NBIG_C, B, D), jnp.float32),
              jax.ShapeDtypeStruct((NBIG_P, P_PAD, D), jnp.float32)),
    mesh=plsc.VectorSubcoreMesh(core_axis_name="c", subcore_axis_name="s",
                                num_cores=NC, num_subcores=NS),
    scratch_types=[
        pltpu.VMEM((NBIG_C, CB), jnp.int32),
        pltpu.VMEM((NBIG_P, PB), jnp.int32),
        pltpu.VMEM((NBIG_C, CB, D), jnp.float32),
        pltpu.VMEM((NCH, NBIG_P, CHUNK, D), jnp.float32),
        pltpu.SemaphoreType.DMA,
        pltpu.SemaphoreType.DMA,
    ],
    compiler_params=pltpu.CompilerParams(use_tc_tiling_on_sc=False),
)(_sc_gather_body)


def _cid_body(cid_smem, cid_v, tabT, out_ref, blk, sem):
    def fire(i, carry):
        k = cid_smem[i]
        tc = pl.multiple_of((k // 128) * 128, 128)
        pltpu.make_async_copy(tabT.at[:, pl.ds(tc, 128)], blk.at[i], sem).start()
        return carry

    lax.fori_loop(0, B, fire, 0)

    def drain(i, carry):
        pltpu.make_async_copy(tabT.at[:, pl.ds(0, 128)], blk.at[i], sem).wait()
        return carry

    lax.fori_loop(0, B, drain, 0)

    lane = cid_v[...] % 128
    iota = lax.broadcasted_iota(jnp.int32, (1, 128), 1)
    for c0 in range(0, B, 128):
        oh = (lane[c0:c0 + 128][:, None] == iota).astype(jnp.float32)
        out_ref[c0:c0 + 128, :] = jnp.sum(
            blk[c0:c0 + 128] * oh[:, None, :], axis=2)


def _cid_gather(cid, tabT):
    return pl.pallas_call(
        _cid_body,
        in_specs=[
            pl.BlockSpec(memory_space=pltpu.SMEM),
            pl.BlockSpec(memory_space=pltpu.VMEM),
            pl.BlockSpec(memory_space=pl.ANY),
        ],
        out_specs=pl.BlockSpec(memory_space=pltpu.VMEM),
        out_shape=jax.ShapeDtypeStruct((B, D), jnp.float32),
        scratch_shapes=[
            pltpu.VMEM((B, D, 128), jnp.float32),
            pltpu.SemaphoreType.DMA,
        ],
        compiler_params=pltpu.CompilerParams(
            vmem_limit_bytes=100 * 1024 * 1024),
    )(cid, cid, tabT)


def _mm_body(c0_ref, cbig_ref, pbig_ref, cidx_ref, pidx_ref, *rest):
    small_refs = rest[:NSM]
    o_ref = rest[NSM]
    csm = rest[NSM + 1]
    j = pl.program_id(0)

    @pl.when(j == 0)
    def _():
        for t in range(NSM):
            ohc = (cidx_ref[t][:, None] ==
                   lax.broadcasted_iota(jnp.int32, (B, SMALL_V[t]), 1))
            csm[t] = lax.dot_general(
                ohc.astype(jnp.bfloat16),
                small_refs[t][...].astype(jnp.bfloat16),
                dimension_numbers=(((1,), (0,)), ((), ())),
                preferred_element_type=jnp.float32,
            ).astype(jnp.bfloat16)

    def pdot(pm, cm):
        return lax.dot_general(
            pm, cm, dimension_numbers=(((1,), (1,)), ((), ())),
            preferred_element_type=jnp.float32)

    acc = pdot(pbig_ref[0].astype(jnp.bfloat16), c0_ref[...].astype(jnp.bfloat16))
    acc += pdot(pbig_ref[1].astype(jnp.bfloat16), cbig_ref[0].astype(jnp.bfloat16))
    acc += pdot(pbig_ref[2].astype(jnp.bfloat16), cbig_ref[1].astype(jnp.bfloat16))
    for t in range(NSM):
        ohp = (pidx_ref[t][:, None] ==
               lax.broadcasted_iota(jnp.int32, (PN, SMALL_V[t]), 1))
        pt = lax.dot_general(
            ohp.astype(jnp.bfloat16), small_refs[t][...].astype(jnp.bfloat16),
            dimension_numbers=(((1,), (0,)), ((), ())),
            preferred_element_type=jnp.float32,
        ).astype(jnp.bfloat16)
        acc += pdot(pt, csm[t])
    o_ref[...] = acc


def _matmul(cfeat0, cfeat_big, pfeat_big, cidx_s, pidx_s, smalls):
    return pl.pallas_call(
        _mm_body,
        grid=(P_PAD // PN,),
        in_specs=[
            pl.BlockSpec((B, D), lambda j: (0, 0)),
            pl.BlockSpec((NBIG_C, B, D), lambda j: (0, 0, 0)),
            pl.BlockSpec((NBIG_P, PN, D), lambda j: (0, j, 0)),
            pl.BlockSpec((NSM, B), lambda j: (0, 0)),
            pl.BlockSpec((NSM, PN), lambda j: (0, j)),
        ] + [pl.BlockSpec((v, D), lambda j: (0, 0)) for v in SMALL_V],
        out_specs=pl.BlockSpec((PN, B), lambda j: (j, 0)),
        out_shape=jax.ShapeDtypeStruct((P, B), jnp.float32),
        scratch_shapes=[pltpu.VMEM((NSM, B, D), jnp.bfloat16)],
        compiler_params=pltpu.CompilerParams(
            vmem_limit_bytes=100 * 1024 * 1024),
    )(cfeat0, cfeat_big, pfeat_big, cidx_s, pidx_s, *smalls)


def kernel(Customer_data, Product_data, customer_table, product_table,
           price_table, age_table, colour_table, department_table,
           prod_name_table, sales_channel_table, season_table, day_table,
           month_table, year_table, club_table, fashion_table, postal_table,
           graphical_table):
    cdat = Customer_data.astype(jnp.int32)
    pdat = jnp.pad(Product_data.astype(jnp.int32), ((0, P_PAD - P), (0, 0)))

    cidx = cdat[:, (4, 11)].T.reshape(NBIG_C, NW, CB).transpose(1, 0, 2)
    pidx = pdat[:, (0, 4, 11)].T.reshape(NBIG_P, NW, PB).transpose(1, 0, 2)

    cfeat_big, pfeat_big = _sc_gather(cidx, pidx, product_table,
                                      postal_table, prod_name_table)
    cfeat0 = _cid_gather(cdat[:, 0], customer_table.T)

    smalls = (club_table, fashion_table, age_table, price_table,
              sales_channel_table, season_table, day_table, month_table,
              year_table, graphical_table, colour_table, department_table)
    cidx_s = cdat[:, SMALL_COLS].T
    pidx_s = pdat[:, SMALL_COLS].T
    return _matmul(cfeat0, cfeat_big, pfeat_big, cidx_s, pidx_s, smalls).T
